# Initial kernel scaffold; baseline (speedup 1.0000x reference)
import jax
import jax.numpy as jnp
from jax.experimental import pallas as pl


def kernel(z, codebooks):
    raise NotImplementedError("write your pallas kernel here")


# speedup vs baseline: 2.4009x; 2.4009x over previous
import jax
import jax.numpy as jnp
from jax import lax
from jax.experimental import pallas as pl

_NCB = 4
_K = 512
_D = 64
_TT = 128


def _rvq_block(z_ref, cbt_ref, qsum_ref, inds_ref, loss_ref):
    i = pl.program_id(0)
    r = z_ref[...]
    lane_k = lax.broadcasted_iota(jnp.int32, (_TT, _K), 1)
    lane128 = lax.broadcasted_iota(jnp.int32, (_TT, 128), 1)
    ind_tile = jnp.zeros((_TT, 128), jnp.int32)
    qacc = jnp.zeros((_TT, _D), jnp.float32)
    loss_val = jnp.float32(0.0)
    for c in range(_NCB):
        wt = cbt_ref[c]
        rt = r.T
        diff = rt[:, :, None] - wt[:, None, :]
        sq = diff * diff
        x = sq.reshape(8, 8, _TT, _K)
        x = x[:, 0:4] + x[:, 4:8]
        x = x[:, 0:2] + x[:, 2:4]
        x = x[:, 0] + x[:, 1]
        dist = x[0]
        for g in range(1, 8):
            dist = dist + x[g]
        m = jnp.min(dist, axis=1, keepdims=True)
        cand = jnp.where(dist == m, lane_k, _K)
        idx = jnp.min(cand, axis=1, keepdims=True)
        oh = (lane_k == idx).astype(jnp.float32)
        zq = lax.dot_general(
            oh, wt, (((1,), (1,)), ((), ())),
            precision=lax.Precision.HIGHEST,
            preferred_element_type=jnp.float32)
        t = zq - r
        u = r + t
        loss_val = loss_val + jnp.sum(t * t)
        qacc = qacc + u
        r = r - u
        ind_tile = jnp.where(lane128 == c,
                             jnp.broadcast_to(idx, (_TT, 128)), ind_tile)
    qsum_ref[...] = qacc
    inds_ref[...] = ind_tile

    @pl.when(i == 0)
    def _():
        loss_ref[...] = jnp.zeros_like(loss_ref)

    loss_ref[...] += jnp.full((8, 128), loss_val, jnp.float32)


def kernel(z, codebooks):
    B, N, D = z.shape
    T = B * N
    zf = z.reshape(T, D)
    cbt = jnp.transpose(codebooks, (0, 2, 1))
    qsum, indsw, lossw = pl.pallas_call(
        _rvq_block,
        grid=(T // _TT,),
        in_specs=[
            pl.BlockSpec((_TT, _D), lambda i: (i, 0)),
            pl.BlockSpec((_NCB, _D, _K), lambda i: (0, 0, 0)),
        ],
        out_specs=[
            pl.BlockSpec((_TT, _D), lambda i: (i, 0)),
            pl.BlockSpec((_TT, 128), lambda i: (i, 0)),
            pl.BlockSpec((8, 128), lambda i: (0, 0)),
        ],
        out_shape=[
            jax.ShapeDtypeStruct((T, _D), jnp.float32),
            jax.ShapeDtypeStruct((T, 128), jnp.int32),
            jax.ShapeDtypeStruct((8, 128), jnp.float32),
        ],
    )(zf, cbt)
    quant_sum = qsum.reshape(B, N, D)
    inds = indsw[:, :_NCB].reshape(B, N, _NCB).transpose(0, 2, 1)
    total_loss = lossw[0, 0] * jnp.float32(2.0 / (B * N * D))
    return quant_sum, inds, total_loss
